# baseline (speedup 1.0000x reference)
import functools

import jax
import jax.numpy as jnp
from jax import lax
from jax.experimental import pallas as pl
from jax.experimental.pallas import tpu as pltpu
from jax.experimental.pallas import tpu_sc as plsc

_NUM = 8192
_DIM = 64
_TOK = 8192
_TBLK = 4096
_SBLK = 2048
_NS = _NUM // _SBLK


def _vq_tc_kernel(z_ref, w_ref, zsq_ref, wsq_ref, idx_ref):
    zf = z_ref[...]
    zsq = zsq_ref[...]
    zb = (2.0 * zf).astype(jnp.bfloat16)

    a_val = jnp.full((_TBLK, 1), jnp.inf, jnp.float32)
    a_idx = jnp.zeros((_TBLK, 1), jnp.int32)
    iota = jax.lax.broadcasted_iota(jnp.int32, (_TBLK, _SBLK), 1)
    for s in range(_NS):
        ws = w_ref[s * _SBLK:(s + 1) * _SBLK, :]
        wsqs = wsq_ref[:, s * _SBLK:(s + 1) * _SBLK]
        p = jax.lax.dot_general(zb, ws, (((1,), (1,)), ((), ())),
                                preferred_element_type=jnp.float32)
        dist = (zsq - p) + wsqs
        m = jnp.min(dist, axis=1, keepdims=True)
        li = jnp.min(jnp.where(dist == m, iota, _NUM), axis=1,
                     keepdims=True) + s * _SBLK
        keep = a_val <= m
        a_idx = jnp.where(keep, a_idx, li)
        a_val = jnp.where(keep, a_val, m).astype(jnp.bfloat16).astype(jnp.float32)
    idx_ref[...] = a_idx


def _st_loss_kernel(z_ref, zq_ref, out_ref, loss_ref):
    zf = z_ref[...]
    zq = zq_ref[:, :_DIM]
    out_ref[...] = zf + (zq - zf)
    diff = zq - zf
    loss_ref[...] = jnp.mean(diff * diff, axis=1, keepdims=True)


def _sc_gather(W_pad, idx):
    info = plsc.get_sparse_core_info()
    nw = info.num_cores * info.num_subcores
    b_per_w = _TOK // nw
    mesh = plsc.VectorSubcoreMesh(core_axis_name="c", subcore_axis_name="s")

    @functools.partial(
        pl.kernel, mesh=mesh,
        out_type=jax.ShapeDtypeStruct((_TOK, 128), jnp.float32),
        scratch_types=[
            pltpu.VMEM((b_per_w,), jnp.int32),
            pltpu.VMEM((b_per_w, 128), jnp.float32),
            pltpu.SemaphoreType.DMA,
        ],
    )
    def gather_kernel(table_hbm, idx_hbm, out_hbm, idx_v, rows_v, sem):
        wid = lax.axis_index("s") * info.num_cores + lax.axis_index("c")
        base = wid * b_per_w
        pltpu.sync_copy(idx_hbm.at[pl.ds(base, b_per_w)], idx_v)
        pltpu.async_copy(table_hbm.at[idx_v], rows_v, sem).wait()
        pltpu.sync_copy(rows_v, out_hbm.at[pl.ds(base, b_per_w)])

    return gather_kernel(W_pad, idx)


def kernel(z, W):
    zf = z.reshape(-1, _DIM)
    zsq = jnp.sum(zf ** 2, axis=1, keepdims=True)
    wsq = jnp.sum(W ** 2, axis=1)[None, :]
    idx = pl.pallas_call(
        _vq_tc_kernel,
        grid=(_TOK // _TBLK,),
        compiler_params=pltpu.CompilerParams(
            dimension_semantics=("parallel",)),
        in_specs=[
            pl.BlockSpec((_TBLK, _DIM), lambda i: (i, 0)),
            pl.BlockSpec((_NUM, _DIM), lambda i: (0, 0)),
            pl.BlockSpec((_TBLK, 1), lambda i: (i, 0)),
            pl.BlockSpec((1, _NUM), lambda i: (0, 0)),
        ],
        out_specs=pl.BlockSpec((_TBLK, 1), lambda i: (i, 0)),
        out_shape=jax.ShapeDtypeStruct((_TOK, 1), jnp.int32),
    )(zf, W, zsq, wsq)

    idx_flat = idx.reshape(-1)
    W_pad = jnp.pad(W, ((0, 0), (0, 128 - _DIM)))
    zq = _sc_gather(W_pad, idx_flat)

    zq_st, loss = pl.pallas_call(
        _st_loss_kernel,
        grid=(_TOK // 1024,),
        compiler_params=pltpu.CompilerParams(
            dimension_semantics=("parallel",)),
        in_specs=[
            pl.BlockSpec((1024, _DIM), lambda i: (i, 0)),
            pl.BlockSpec((1024, 128), lambda i: (i, 0)),
        ],
        out_specs=[
            pl.BlockSpec((1024, _DIM), lambda i: (i, 0)),
            pl.BlockSpec((1024, 1), lambda i: (i, 0)),
        ],
        out_shape=[
            jax.ShapeDtypeStruct((_TOK, _DIM), jnp.float32),
            jax.ShapeDtypeStruct((_TOK, 1), jnp.float32),
        ],
    )(zf, zq)

    z_q_st = zq_st.reshape(z.shape)
    quant_l = loss.reshape(z.shape[:-1])
    commit_l = loss.reshape(z.shape[:-1])
    idx_out = idx.reshape(z.shape[:-1] + (1,))
    return (z_q_st, quant_l, commit_l, idx_out)


# speedup vs baseline: 1.2730x; 1.2730x over previous
import functools

import jax
import jax.numpy as jnp
from jax import lax
from jax.experimental import pallas as pl
from jax.experimental.pallas import tpu as pltpu
from jax.experimental.pallas import tpu_sc as plsc

_NUM = 8192
_DIM = 64
_TOK = 8192
_TBLK = 2048
_SBLK = 2048
_NS = _NUM // _SBLK


def _vq_tc_kernel(z_ref, w_ref, zsq_ref, wsq_ref, idx_ref):
    zf = z_ref[...]
    zsq = zsq_ref[...]
    zb = (2.0 * zf).astype(jnp.bfloat16)

    a_val = jnp.full((_TBLK, 1), jnp.inf, jnp.float32)
    a_idx = jnp.zeros((_TBLK, 1), jnp.int32)
    iota = jax.lax.broadcasted_iota(jnp.int32, (_TBLK, _SBLK), 1)
    for s in range(_NS):
        ws = w_ref[s * _SBLK:(s + 1) * _SBLK, :]
        wsqs = wsq_ref[:, s * _SBLK:(s + 1) * _SBLK]
        p = jax.lax.dot_general(zb, ws, (((1,), (1,)), ((), ())),
                                preferred_element_type=jnp.float32)
        dist = (zsq - p) + wsqs
        m = jnp.min(dist, axis=1, keepdims=True)
        li = jnp.min(jnp.where(dist == m, iota, _NUM), axis=1,
                     keepdims=True) + s * _SBLK
        keep = a_val <= m
        a_idx = jnp.where(keep, a_idx, li)
        a_val = jnp.where(keep, a_val, m).astype(jnp.bfloat16).astype(jnp.float32)
    idx_ref[...] = a_idx


def _st_loss_kernel(z_ref, zq_ref, out_ref, loss_ref):
    zf = z_ref[...]
    zq = zq_ref[:, :_DIM]
    out_ref[...] = zf + (zq - zf)
    diff = zq - zf
    loss_ref[...] = jnp.mean(diff * diff, axis=1, keepdims=True)


def _sc_gather(W_pad, idx):
    info = plsc.get_sparse_core_info()
    nw = info.num_cores * info.num_subcores
    b_per_w = _TOK // nw
    mesh = plsc.VectorSubcoreMesh(core_axis_name="c", subcore_axis_name="s")

    @functools.partial(
        pl.kernel, mesh=mesh,
        out_type=jax.ShapeDtypeStruct((_TOK, 128), jnp.float32),
        scratch_types=[
            pltpu.VMEM((b_per_w,), jnp.int32),
            pltpu.VMEM((b_per_w, 128), jnp.float32),
            pltpu.SemaphoreType.DMA,
        ],
    )
    def gather_kernel(table_hbm, idx_hbm, out_hbm, idx_v, rows_v, sem):
        wid = lax.axis_index("s") * info.num_cores + lax.axis_index("c")
        base = wid * b_per_w
        pltpu.sync_copy(idx_hbm.at[pl.ds(base, b_per_w)], idx_v)
        pltpu.async_copy(table_hbm.at[idx_v], rows_v, sem).wait()
        pltpu.sync_copy(rows_v, out_hbm.at[pl.ds(base, b_per_w)])

    return gather_kernel(W_pad, idx)


def kernel(z, W):
    zf = z.reshape(-1, _DIM)
    zsq = jnp.sum(zf ** 2, axis=1, keepdims=True)
    wsq = jnp.sum(W ** 2, axis=1)[None, :]
    idx = pl.pallas_call(
        _vq_tc_kernel,
        grid=(_TOK // _TBLK,),
        compiler_params=pltpu.CompilerParams(
            dimension_semantics=("parallel",)),
        in_specs=[
            pl.BlockSpec((_TBLK, _DIM), lambda i: (i, 0)),
            pl.BlockSpec((_NUM, _DIM), lambda i: (0, 0)),
            pl.BlockSpec((_TBLK, 1), lambda i: (i, 0)),
            pl.BlockSpec((1, _NUM), lambda i: (0, 0)),
        ],
        out_specs=pl.BlockSpec((_TBLK, 1), lambda i: (i, 0)),
        out_shape=jax.ShapeDtypeStruct((_TOK, 1), jnp.int32),
    )(zf, W, zsq, wsq)

    idx_flat = idx.reshape(-1)
    W_pad = jnp.pad(W, ((0, 0), (0, 128 - _DIM)))
    zq = _sc_gather(W_pad, idx_flat)

    zq_st, loss = pl.pallas_call(
        _st_loss_kernel,
        grid=(_TOK // 1024,),
        compiler_params=pltpu.CompilerParams(
            dimension_semantics=("parallel",)),
        in_specs=[
            pl.BlockSpec((1024, _DIM), lambda i: (i, 0)),
            pl.BlockSpec((1024, 128), lambda i: (i, 0)),
        ],
        out_specs=[
            pl.BlockSpec((1024, _DIM), lambda i: (i, 0)),
            pl.BlockSpec((1024, 1), lambda i: (i, 0)),
        ],
        out_shape=[
            jax.ShapeDtypeStruct((_TOK, _DIM), jnp.float32),
            jax.ShapeDtypeStruct((_TOK, 1), jnp.float32),
        ],
    )(zf, zq)

    z_q_st = zq_st.reshape(z.shape)
    quant_l = loss.reshape(z.shape[:-1])
    commit_l = loss.reshape(z.shape[:-1])
    idx_out = idx.reshape(z.shape[:-1] + (1,))
    return (z_q_st, quant_l, commit_l, idx_out)
